# baseline (speedup 1.0000x reference)
import jax
import jax.numpy as jnp
from jax.experimental import pallas as pl

F32 = jnp.float32
BF16 = jnp.bfloat16
KB = 256
TM = 256


def _dot(a, b):
    return jnp.dot(a, b, preferred_element_type=F32)


def _k0_body(wd1, we1, wd2, we2, m1, m2):
    m1[...] = _dot(wd1[...], we1[...])
    m2[...] = _dot(wd2[...], we2[...])


def _attention(a, b, w, u):
    sa = _dot(jnp.tanh(_dot(a, w)), u) + 1e-6
    sb = _dot(jnp.tanh(_dot(b, w)), u) + 1e-6
    m = jnp.maximum(sa, sb)
    ea = jnp.exp(sa - m)
    eb = jnp.exp(sb - m)
    inv = 1.0 / (ea + eb)
    aa = ea * inv
    ab = eb * inv
    return a * aa + b * ab, jnp.concatenate([aa, ab], axis=1)


def _discriminator(x, w1t, b1, w2t, b2, w3t, b3):
    h = _dot(x, w1t) + b1
    h = jnp.where(h >= 0, h, 0.01 * h)
    h = _dot(h, w2t) + b2
    h = jnp.where(h >= 0, h, 0.01 * h)
    return jax.nn.sigmoid(_dot(h, w3t) + b3)


def _agg_body(f, aa, ab, we, eao, ebo):
    kb = pl.program_id(0)
    tk = _dot(f[...], we[...])
    pa = _dot(aa[...], tk)
    pb = _dot(ab[...], tk)

    @pl.when(kb == 0)
    def _init():
        eao[...] = pa
        ebo[...] = pb

    @pl.when(kb != 0)
    def _accum():
        eao[...] = eao[...] + pa
        ebo[...] = ebo[...] + pb


def _att_body(e11, e12, e21, e22, wo1, uo1, wo2, uo2, woc, uoc,
              w1t, b1, w2t, b2, w3t, b3,
              e1w, e2w, comb, al1, al2, al12, s1, s2, ce1o, ce2o):
    E1, A1 = _attention(e11[...], e12[...], wo1[...], uo1[...])
    E2, A2 = _attention(e21[...], e22[...], wo2[...], uo2[...])
    C, A12 = _attention(E1, E2, woc[...], uoc[...])
    e1w[...] = E1
    e2w[...] = E2
    comb[...] = C
    al1[...] = A1
    al2[...] = A2
    al12[...] = A12
    s1[...] = _discriminator(E1, w1t[...], b1[...], w2t[...], b2[...],
                             w3t[...], b3[...])
    s2[...] = _discriminator(E2, w1t[...], b1[...], w2t[...], b2[...],
                             w3t[...], b3[...])
    ce1o[...] = jnp.concatenate([C, E1], axis=1).astype(BF16)
    ce2o[...] = jnp.concatenate([C, E2], axis=1).astype(BF16)


def _k3_body(a11, a21, ce1, ce2, wd1, wd2, m1, m2,
             r1, r2, z1, z2, a11h, a21h):
    a11v = a11[...]
    a21v = a21[...]
    a11h[...] = a11v.astype(BF16)
    a21h[...] = a21v.astype(BF16)
    R1 = _dot(a11v, ce2[...].astype(F32))
    R2 = _dot(a21v, ce1[...].astype(F32))
    r1[...] = _dot(R1[:, :64], wd1[...])
    r2[...] = _dot(R2[:, :64], wd2[...])
    z1[...] = _dot(R2[:, 64:], m2[...])
    z2[...] = _dot(R1[:, 64:], m1[...])


def _k4_body(a11h, a21h, z1, z2, l1, l2):
    l1[...] = _dot(a21h[...], z1[...].astype(BF16))
    l2[...] = _dot(a11h[...], z2[...].astype(BF16))


def _full(shape):
    return pl.BlockSpec(shape, lambda i: (0,) * len(shape))


def kernel(feat1, feat2, adj1_1, adj1_2, adj2_1, adj2_2,
           W_enc1, W_dec1, W_enc2, W_dec2,
           wo1, uo1, wo2, uo2, woc, uoc,
           dW1, db1, dW2, db2, dW3, db3):
    N, D1 = feat1.shape
    D2 = feat2.shape[1]
    O1 = W_enc1.shape[1]
    O2 = W_enc2.shape[1]
    HID = dW1.shape[0]
    KS = N // KB
    G = N // TM

    M1, M2 = pl.pallas_call(
        _k0_body,
        out_shape=[jax.ShapeDtypeStruct((O1, O1), F32),
                   jax.ShapeDtypeStruct((O2, O2), F32)],
    )(W_dec1, W_enc1, W_dec2, W_enc2)

    rowsK = lambda cols: pl.BlockSpec((KB, cols), lambda i: (i, 0))
    colsK = lambda: pl.BlockSpec((N, KB), lambda i: (0, i))

    def _agg_call(f, aa, ab, we, D, O):
        return pl.pallas_call(
            _agg_body,
            grid=(KS,),
            in_specs=[rowsK(D), colsK(), colsK(), _full((D, O))],
            out_specs=[_full((N, O)), _full((N, O))],
            out_shape=[jax.ShapeDtypeStruct((N, O), F32),
                       jax.ShapeDtypeStruct((N, O), F32)],
        )(f, aa, ab, we)

    e11, e12 = _agg_call(feat1, adj1_1, adj1_2, W_enc1, D1, O1)
    e21, e22 = _agg_call(feat2, adj2_1, adj2_2, W_enc2, D2, O2)

    rowsT = lambda cols: pl.BlockSpec((TM, cols), lambda i: (i, 0))

    (E1w, E2w, C, al1, al2, al12, s1, s2, ce1, ce2) = pl.pallas_call(
        _att_body,
        grid=(G,),
        in_specs=[rowsT(O1), rowsT(O1), rowsT(O2), rowsT(O2),
                  _full((O1, O1)), _full((O1, 1)),
                  _full((O2, O2)), _full((O2, 1)),
                  _full((O1, O2)), _full((O2, 1)),
                  _full((O1, HID)), _full((1, HID)),
                  _full((HID, 2 * HID)), _full((1, 2 * HID)),
                  _full((2 * HID, 1)), _full((1, 1))],
        out_specs=[rowsT(O1), rowsT(O2), rowsT(O2),
                   rowsT(2), rowsT(2), rowsT(2), rowsT(1), rowsT(1),
                   rowsT(2 * O1), rowsT(2 * O2)],
        out_shape=[jax.ShapeDtypeStruct((N, O1), F32),
                   jax.ShapeDtypeStruct((N, O2), F32),
                   jax.ShapeDtypeStruct((N, O2), F32),
                   jax.ShapeDtypeStruct((N, 2), F32),
                   jax.ShapeDtypeStruct((N, 2), F32),
                   jax.ShapeDtypeStruct((N, 2), F32),
                   jax.ShapeDtypeStruct((N, 1), F32),
                   jax.ShapeDtypeStruct((N, 1), F32),
                   jax.ShapeDtypeStruct((N, 2 * O1), BF16),
                   jax.ShapeDtypeStruct((N, 2 * O2), BF16)],
    )(e11, e12, e21, e22, wo1, uo1, wo2, uo2, woc, uoc,
      dW1.T, db1.reshape(1, -1), dW2.T, db2.reshape(1, -1),
      dW3.T, db3.reshape(1, 1))

    r1, r2, z1, z2, a11h, a21h = pl.pallas_call(
        _k3_body,
        grid=(G,),
        in_specs=[rowsT(N), rowsT(N),
                  _full((N, 2 * O1)), _full((N, 2 * O2)),
                  _full((O1, D1)), _full((O2, D2)),
                  _full((O1, O1)), _full((O2, O2))],
        out_specs=[rowsT(D1), rowsT(D2), rowsT(O2), rowsT(O1),
                   rowsT(N), rowsT(N)],
        out_shape=[jax.ShapeDtypeStruct((N, D1), F32),
                   jax.ShapeDtypeStruct((N, D2), F32),
                   jax.ShapeDtypeStruct((N, O2), F32),
                   jax.ShapeDtypeStruct((N, O1), F32),
                   jax.ShapeDtypeStruct((N, N), BF16),
                   jax.ShapeDtypeStruct((N, N), BF16)],
    )(adj1_1, adj2_1, ce1, ce2, W_dec1, W_dec2, M1, M2)

    l1, l2 = pl.pallas_call(
        _k4_body,
        grid=(G,),
        in_specs=[rowsT(N), rowsT(N), _full((N, O2)), _full((N, O1))],
        out_specs=[rowsT(O2), rowsT(O1)],
        out_shape=[jax.ShapeDtypeStruct((N, O2), F32),
                   jax.ShapeDtypeStruct((N, O1), F32)],
    )(a11h, a21h, z1, z2)

    return (E1w, E2w, C, l1, l2, r1, r2, al1, al2, al12,
            jnp.squeeze(s1, axis=1), jnp.squeeze(s2, axis=1))


# speedup vs baseline: 1.1062x; 1.1062x over previous
import jax
import jax.numpy as jnp
from jax.experimental import pallas as pl

F32 = jnp.float32
BF16 = jnp.bfloat16
TM = 256


def _dot(a, b):
    return jnp.dot(a, b, preferred_element_type=F32)


def _k0_body(wd1, we1, wd2, we2, m1, m2):
    m1[...] = _dot(wd1[...], we1[...])
    m2[...] = _dot(wd2[...], we2[...])


def _k1_body(f1, f2, we1, we2, t1, t2):
    t1[...] = _dot(f1[...], we1[...])
    t2[...] = _dot(f2[...], we2[...])


def _attention(a, b, w, u):
    sa = _dot(jnp.tanh(_dot(a, w)), u) + 1e-6
    sb = _dot(jnp.tanh(_dot(b, w)), u) + 1e-6
    m = jnp.maximum(sa, sb)
    ea = jnp.exp(sa - m)
    eb = jnp.exp(sb - m)
    inv = 1.0 / (ea + eb)
    aa = ea * inv
    ab = eb * inv
    return a * aa + b * ab, jnp.concatenate([aa, ab], axis=1)


def _discriminator(x, w1t, b1, w2t, b2, w3t, b3):
    h = _dot(x, w1t) + b1
    h = jnp.where(h >= 0, h, 0.01 * h)
    h = _dot(h, w2t) + b2
    h = jnp.where(h >= 0, h, 0.01 * h)
    return jax.nn.sigmoid(_dot(h, w3t) + b3)


def _k2_body(a11, a12, a21, a22, t1, t2, wo1, uo1, wo2, uo2, woc, uoc,
             w1t, b1, w2t, b2, w3t, b3,
             e1w, e2w, comb, al1, al2, al12, s1, s2,
             a11h, a21h, ce1, ce2):
    t1v = t1[...]
    t2v = t2[...]
    a11v = a11[...]
    a21v = a21[...]
    e11 = _dot(a11v, t1v)
    e12 = _dot(a12[...], t1v)
    e21 = _dot(a21v, t2v)
    e22 = _dot(a22[...], t2v)
    E1, A1 = _attention(e11, e12, wo1[...], uo1[...])
    E2, A2 = _attention(e21, e22, wo2[...], uo2[...])
    C, A12 = _attention(E1, E2, woc[...], uoc[...])
    e1w[...] = E1
    e2w[...] = E2
    comb[...] = C
    al1[...] = A1
    al2[...] = A2
    al12[...] = A12
    s1[...] = _discriminator(E1, w1t[...], b1[...], w2t[...], b2[...],
                             w3t[...], b3[...])
    s2[...] = _discriminator(E2, w1t[...], b1[...], w2t[...], b2[...],
                             w3t[...], b3[...])
    a11h[...] = a11v.astype(jnp.bfloat16)
    a21h[...] = a21v.astype(jnp.bfloat16)
    ce1[...] = jnp.concatenate([C, E1], axis=1).astype(jnp.bfloat16)
    ce2[...] = jnp.concatenate([C, E2], axis=1).astype(jnp.bfloat16)


def _k3_body(a11h, a21h, ce1, ce2, wd1, wd2, m1, m2, r1, r2, z1, z2):
    R1 = _dot(a11h[...], ce2[...])
    R2 = _dot(a21h[...], ce1[...])
    r1[...] = _dot(R1[:, :64], wd1[...])
    r2[...] = _dot(R2[:, :64], wd2[...])
    z1[...] = _dot(R2[:, 64:], m2[...])
    z2[...] = _dot(R1[:, 64:], m1[...])


def _k4_body(a11h, a21h, z1, z2, l1, l2):
    l1[...] = _dot(a21h[...], z1[...].astype(jnp.bfloat16))
    l2[...] = _dot(a11h[...], z2[...].astype(jnp.bfloat16))


def _full(shape):
    return pl.BlockSpec(shape, lambda i: (0,) * len(shape))


def _rows(cols):
    return pl.BlockSpec((TM, cols), lambda i: (i, 0))


def kernel(feat1, feat2, adj1_1, adj1_2, adj2_1, adj2_2,
           W_enc1, W_dec1, W_enc2, W_dec2,
           wo1, uo1, wo2, uo2, woc, uoc,
           dW1, db1, dW2, db2, dW3, db3):
    N, D1 = feat1.shape
    D2 = feat2.shape[1]
    O1 = W_enc1.shape[1]
    O2 = W_enc2.shape[1]
    grid = (N // TM,)

    M1, M2 = pl.pallas_call(
        _k0_body,
        out_shape=[jax.ShapeDtypeStruct((O1, O1), F32),
                   jax.ShapeDtypeStruct((O2, O2), F32)],
    )(W_dec1, W_enc1, W_dec2, W_enc2)

    t1, t2 = pl.pallas_call(
        _k1_body,
        grid=grid,
        in_specs=[_rows(D1), _rows(D2), _full((D1, O1)), _full((D2, O2))],
        out_specs=[_rows(O1), _rows(O2)],
        out_shape=[jax.ShapeDtypeStruct((N, O1), F32),
                   jax.ShapeDtypeStruct((N, O2), F32)],
    )(feat1, feat2, W_enc1, W_enc2)

    HID = dW1.shape[0]
    k2_out = pl.pallas_call(
        _k2_body,
        grid=grid,
        in_specs=[_rows(N), _rows(N), _rows(N), _rows(N),
                  _full((N, O1)), _full((N, O2)),
                  _full((O1, O1)), _full((O1, 1)),
                  _full((O2, O2)), _full((O2, 1)),
                  _full((O1, O2)), _full((O2, 1)),
                  _full((O1, HID)), _full((1, HID)),
                  _full((HID, 2 * HID)), _full((1, 2 * HID)),
                  _full((2 * HID, 1)), _full((1, 1))],
        out_specs=[_rows(O1), _rows(O2), _rows(O2),
                   _rows(2), _rows(2), _rows(2), _rows(1), _rows(1),
                   _rows(N), _rows(N), _rows(2 * O1), _rows(2 * O2)],
        out_shape=[jax.ShapeDtypeStruct((N, O1), F32),
                   jax.ShapeDtypeStruct((N, O2), F32),
                   jax.ShapeDtypeStruct((N, O2), F32),
                   jax.ShapeDtypeStruct((N, 2), F32),
                   jax.ShapeDtypeStruct((N, 2), F32),
                   jax.ShapeDtypeStruct((N, 2), F32),
                   jax.ShapeDtypeStruct((N, 1), F32),
                   jax.ShapeDtypeStruct((N, 1), F32),
                   jax.ShapeDtypeStruct((N, N), BF16),
                   jax.ShapeDtypeStruct((N, N), BF16),
                   jax.ShapeDtypeStruct((N, 2 * O1), BF16),
                   jax.ShapeDtypeStruct((N, 2 * O2), BF16)],
    )(adj1_1, adj1_2, adj2_1, adj2_2, t1, t2,
      wo1, uo1, wo2, uo2, woc, uoc,
      dW1.T, db1.reshape(1, -1), dW2.T, db2.reshape(1, -1),
      dW3.T, db3.reshape(1, 1))
    E1w, E2w, C, al1, al2, al12, s1, s2, a11h, a21h, ce1, ce2 = k2_out

    r1, r2, z1, z2 = pl.pallas_call(
        _k3_body,
        grid=grid,
        in_specs=[_rows(N), _rows(N),
                  _full((N, 2 * O1)), _full((N, 2 * O2)),
                  _full((O1, D1)), _full((O2, D2)),
                  _full((O1, O1)), _full((O2, O2))],
        out_specs=[_rows(D1), _rows(D2), _rows(O2), _rows(O1)],
        out_shape=[jax.ShapeDtypeStruct((N, D1), F32),
                   jax.ShapeDtypeStruct((N, D2), F32),
                   jax.ShapeDtypeStruct((N, O2), F32),
                   jax.ShapeDtypeStruct((N, O1), F32)],
    )(a11h, a21h, ce1, ce2, W_dec1, W_dec2, M1, M2)

    l1, l2 = pl.pallas_call(
        _k4_body,
        grid=grid,
        in_specs=[_rows(N), _rows(N), _full((N, O2)), _full((N, O1))],
        out_specs=[_rows(O2), _rows(O1)],
        out_shape=[jax.ShapeDtypeStruct((N, O2), F32),
                   jax.ShapeDtypeStruct((N, O1), F32)],
    )(a11h, a21h, z1, z2)

    return (E1w, E2w, C, l1, l2, r1, r2, al1, al2, al12,
            jnp.squeeze(s1, axis=1), jnp.squeeze(s2, axis=1))
